# Initial kernel scaffold; baseline (speedup 1.0000x reference)
import jax
import jax.numpy as jnp
from jax.experimental import pallas as pl


def kernel(x, weight):
    raise NotImplementedError("write your pallas kernel here")


# speedup vs baseline: 2.9745x; 2.9745x over previous
import functools
import jax
import jax.numpy as jnp
from jax import lax
from jax.experimental import pallas as pl
from jax.experimental.pallas import tpu as pltpu
from jax.experimental.pallas import tpu_sc as plsc

VOCAB_ = 100000
EMBED_ = 128
CHUNK = 128


def _make_kernel(B, D):
    info = plsc.get_sparse_core_info()
    NC, NS = info.num_cores, info.num_subcores
    NW = NC * NS
    assert B % NW == 0
    b_per_w = B // NW
    assert b_per_w % CHUNK == 0
    n_chunks = b_per_w // CHUNK

    mesh = plsc.VectorSubcoreMesh(core_axis_name="c", subcore_axis_name="s")

    @functools.partial(
        pl.kernel,
        mesh=mesh,
        out_type=jax.ShapeDtypeStruct((B, D), jnp.float32),
        scratch_types=[
            pltpu.VMEM((b_per_w,), jnp.int32),
            pltpu.VMEM((CHUNK, D), jnp.float32),
            pltpu.SemaphoreType.DMA,
        ],
    )
    def k(table_hbm, idx_hbm, out_hbm, idx_v, rows_v, sem):
        wid = lax.axis_index("s") * NC + lax.axis_index("c")
        base = wid * b_per_w
        pltpu.sync_copy(idx_hbm.at[pl.ds(base, b_per_w)], idx_v)

        def body(g, carry):
            off = g * CHUNK
            pltpu.async_copy(
                table_hbm.at[idx_v.at[pl.ds(off, CHUNK)]], rows_v, sem
            ).wait()
            pltpu.sync_copy(rows_v, out_hbm.at[pl.ds(base + off, CHUNK)])
            return carry

        lax.fori_loop(0, n_chunks, body, 0)

    return k


def kernel(x, weight):
    B, S = x.shape
    V, D = weight.shape
    flat = x.reshape(B * S).astype(jnp.int32)
    k = _make_kernel(B * S, D)
    out = k(weight, flat)
    return out.reshape(B, S, D)


# speedup vs baseline: 3.3442x; 1.1243x over previous
import functools
import jax
import jax.numpy as jnp
from jax import lax
from jax.experimental import pallas as pl
from jax.experimental.pallas import tpu as pltpu
from jax.experimental.pallas import tpu_sc as plsc

CHUNK = 128
NBUF = 5


def _make_kernel(B, D):
    info = plsc.get_sparse_core_info()
    NC, NS = info.num_cores, info.num_subcores
    NW = NC * NS
    assert B % NW == 0
    b_per_w = B // NW
    assert b_per_w % (CHUNK * NBUF) == 0
    n_outer = b_per_w // (CHUNK * NBUF)

    mesh = plsc.VectorSubcoreMesh(core_axis_name="c", subcore_axis_name="s")

    @functools.partial(
        pl.kernel,
        mesh=mesh,
        out_type=jax.ShapeDtypeStruct((B, D), jnp.float32),
        scratch_types=(
            [pltpu.VMEM((b_per_w,), jnp.int32)]
            + [pltpu.VMEM((CHUNK, D), jnp.float32) for _ in range(NBUF)]
            + [pltpu.SemaphoreType.DMA, pltpu.SemaphoreType.DMA]
        ),
    )
    def k(table_hbm, idx_hbm, out_hbm, idx_v, *rest):
        rows = rest[:NBUF]
        sem_g, sem_w = rest[NBUF], rest[NBUF + 1]
        wid = lax.axis_index("s") * NC + lax.axis_index("c")
        base = wid * b_per_w
        pltpu.sync_copy(idx_hbm.at[pl.ds(base, b_per_w)], idx_v)

        def gather(off, buf):
            return pltpu.async_copy(
                table_hbm.at[idx_v.at[pl.ds(off, CHUNK)]], buf, sem_g
            )

        def write(off, buf):
            return pltpu.async_copy(
                buf, out_hbm.at[pl.ds(base + off, CHUNK)], sem_w
            )

        def wait_write(off, buf):
            pltpu.make_async_copy(
                buf, out_hbm.at[pl.ds(base + off, CHUNK)], sem_w
            ).wait()

        def wait_gather(off, buf):
            pltpu.make_async_copy(
                table_hbm.at[idx_v.at[pl.ds(off, CHUNK)]], buf, sem_g
            ).wait()

        def outer(o, carry):
            g0 = o * NBUF * CHUNK
            for b in range(NBUF):
                off = g0 + b * CHUNK

                @pl.when(o > 0)
                def _():
                    wait_write(off - NBUF * CHUNK, rows[b])

                gather(off, rows[b])
            for b in range(NBUF):
                off = g0 + b * CHUNK
                wait_gather(off, rows[b])
                write(off, rows[b])
            return carry

        lax.fori_loop(0, n_outer, outer, 0)
        for b in range(NBUF):
            wait_write((n_outer - 1) * NBUF * CHUNK + b * CHUNK, rows[b])

    return k


def kernel(x, weight):
    B, S = x.shape
    V, D = weight.shape
    flat = x.reshape(B * S).astype(jnp.int32)
    k = _make_kernel(B * S, D)
    out = k(weight, flat)
    return out.reshape(B, S, D)


# speedup vs baseline: 5.9923x; 1.7918x over previous
import functools
import jax
import jax.numpy as jnp
from jax import lax
from jax.experimental import pallas as pl
from jax.experimental.pallas import tpu as pltpu
from jax.experimental.pallas import tpu_sc as plsc

NBUF = 8


def _make_kernel(B, S, D):
    info = plsc.get_sparse_core_info()
    NC, NS = info.num_cores, info.num_subcores
    NW = NC * NS
    assert B % NW == 0
    rows_per_w = B // NW
    assert rows_per_w % NBUF == 0
    n_outer = rows_per_w // NBUF

    mesh = plsc.VectorSubcoreMesh(core_axis_name="c", subcore_axis_name="s")

    @functools.partial(
        pl.kernel,
        mesh=mesh,
        out_type=jax.ShapeDtypeStruct((B, S, D), jnp.float32),
        scratch_types=(
            [pltpu.VMEM((rows_per_w, S), jnp.int32)]
            + [pltpu.VMEM((S, D), jnp.float32) for _ in range(NBUF)]
            + [pltpu.SemaphoreType.DMA, pltpu.SemaphoreType.DMA]
        ),
    )
    def k(table_hbm, x_hbm, out_hbm, idx_v, *rest):
        bufs = rest[:NBUF]
        sem_g, sem_w = rest[NBUF], rest[NBUF + 1]
        wid = lax.axis_index("s") * NC + lax.axis_index("c")
        base = wid * rows_per_w
        pltpu.sync_copy(x_hbm.at[pl.ds(base, rows_per_w)], idx_v)

        def gather(r, buf):
            return pltpu.async_copy(table_hbm.at[idx_v.at[r]], buf, sem_g)

        def wait_gather(r, buf):
            pltpu.make_async_copy(table_hbm.at[idx_v.at[r]], buf, sem_g).wait()

        def write(r, buf):
            return pltpu.async_copy(buf, out_hbm.at[base + r], sem_w)

        def wait_write(r, buf):
            pltpu.make_async_copy(buf, out_hbm.at[base + r], sem_w).wait()

        def outer(o, carry):
            r0 = o * NBUF
            for b in range(NBUF):
                @pl.when(o > 0)
                def _():
                    wait_write(r0 - NBUF + b, bufs[b])

                gather(r0 + b, bufs[b])
            for b in range(NBUF):
                wait_gather(r0 + b, bufs[b])
                write(r0 + b, bufs[b])
            return carry

        lax.fori_loop(0, n_outer, outer, 0)
        for b in range(NBUF):
            wait_write((n_outer - 1) * NBUF + b, bufs[b])

    return k


def kernel(x, weight):
    B, S = x.shape
    V, D = weight.shape
    k = _make_kernel(B, S, D)
    return k(weight, x.astype(jnp.int32))


# speedup vs baseline: 5.9941x; 1.0003x over previous
import functools
import jax
import jax.numpy as jnp
from jax import lax
from jax.experimental import pallas as pl
from jax.experimental.pallas import tpu as pltpu
from jax.experimental.pallas import tpu_sc as plsc

NBUF = 8


def _make_kernel(B, S, D):
    info = plsc.get_sparse_core_info()
    NC, NS = info.num_cores, info.num_subcores
    NW = NC * NS
    assert B % NW == 0
    rows_per_w = B // NW
    assert rows_per_w % NBUF == 0
    n_outer = rows_per_w // NBUF

    mesh = plsc.VectorSubcoreMesh(core_axis_name="c", subcore_axis_name="s")

    @functools.partial(
        pl.kernel,
        mesh=mesh,
        compiler_params=pltpu.CompilerParams(use_tc_tiling_on_sc=True),
        out_type=jax.ShapeDtypeStruct((B, S, D), jnp.float32),
        scratch_types=(
            [pltpu.VMEM((rows_per_w, S), jnp.int32)]
            + [pltpu.VMEM((S, D), jnp.float32) for _ in range(NBUF)]
            + [pltpu.SemaphoreType.DMA, pltpu.SemaphoreType.DMA]
        ),
    )
    def k(table_hbm, x_hbm, out_hbm, idx_v, *rest):
        bufs = rest[:NBUF]
        sem_g, sem_w = rest[NBUF], rest[NBUF + 1]
        wid = lax.axis_index("s") * NC + lax.axis_index("c")
        base = wid * rows_per_w
        pltpu.sync_copy(x_hbm.at[pl.ds(base, rows_per_w)], idx_v)

        def gather(r, buf):
            return pltpu.async_copy(table_hbm.at[idx_v.at[r]], buf, sem_g)

        def wait_gather(r, buf):
            pltpu.make_async_copy(table_hbm.at[idx_v.at[r]], buf, sem_g).wait()

        def write(r, buf):
            return pltpu.async_copy(buf, out_hbm.at[base + r], sem_w)

        def wait_write(r, buf):
            pltpu.make_async_copy(buf, out_hbm.at[base + r], sem_w).wait()

        def outer(o, carry):
            r0 = o * NBUF
            for b in range(NBUF):
                @pl.when(o > 0)
                def _():
                    wait_write(r0 - NBUF + b, bufs[b])

                gather(r0 + b, bufs[b])
            for b in range(NBUF):
                wait_gather(r0 + b, bufs[b])
                write(r0 + b, bufs[b])
            return carry

        lax.fori_loop(0, n_outer, outer, 0)
        for b in range(NBUF):
            wait_write((n_outer - 1) * NBUF + b, bufs[b])

    return k


def kernel(x, weight):
    B, S = x.shape
    V, D = weight.shape
    k = _make_kernel(B, S, D)
    return k(weight, x.astype(jnp.int32))


# speedup vs baseline: 6.0195x; 1.0042x over previous
import functools
import jax
import jax.numpy as jnp
from jax import lax
from jax.experimental import pallas as pl
from jax.experimental.pallas import tpu as pltpu
from jax.experimental.pallas import tpu_sc as plsc

NBUF = 8


def _make_kernel(B, S, D):
    info = plsc.get_sparse_core_info()
    NC, NS = info.num_cores, info.num_subcores
    NW = NC * NS
    assert B % NW == 0
    rows_per_w = B // NW
    assert rows_per_w % NBUF == 0
    n_outer = rows_per_w // NBUF

    mesh = plsc.VectorSubcoreMesh(core_axis_name="c", subcore_axis_name="s")

    @functools.partial(
        pl.kernel,
        mesh=mesh,
        compiler_params=pltpu.CompilerParams(
            use_tc_tiling_on_sc=True, needs_layout_passes=True
        ),
        out_type=jax.ShapeDtypeStruct((B, S, D), jnp.float32),
        scratch_types=(
            [pltpu.VMEM((rows_per_w, S), jnp.int32)]
            + [pltpu.VMEM((S, D), jnp.float32) for _ in range(NBUF)]
            + [pltpu.SemaphoreType.DMA, pltpu.SemaphoreType.DMA]
        ),
    )
    def k(table_hbm, x_hbm, out_hbm, idx_v, *rest):
        bufs = rest[:NBUF]
        sem_g, sem_w = rest[NBUF], rest[NBUF + 1]
        wid = lax.axis_index("s") * NC + lax.axis_index("c")
        base = wid * rows_per_w
        pltpu.sync_copy(x_hbm.at[pl.ds(base, rows_per_w)], idx_v)

        def gather(r, buf):
            return pltpu.async_copy(table_hbm.at[idx_v.at[r]], buf, sem_g)

        def wait_gather(r, buf):
            pltpu.make_async_copy(table_hbm.at[idx_v.at[r]], buf, sem_g).wait()

        def write(r, buf):
            return pltpu.async_copy(buf, out_hbm.at[base + r], sem_w)

        def wait_write(r, buf):
            pltpu.make_async_copy(buf, out_hbm.at[base + r], sem_w).wait()

        def outer(o, carry):
            r0 = o * NBUF
            for b in range(NBUF):
                @pl.when(o > 0)
                def _():
                    wait_write(r0 - NBUF + b, bufs[b])

                gather(r0 + b, bufs[b])
            for b in range(NBUF):
                wait_gather(r0 + b, bufs[b])
                write(r0 + b, bufs[b])
            return carry

        lax.fori_loop(0, n_outer, outer, 0)
        for b in range(NBUF):
            wait_write((n_outer - 1) * NBUF + b, bufs[b])

    return k


def kernel(x, weight):
    B, S = x.shape
    V, D = weight.shape
    k = _make_kernel(B, S, D)
    return k(weight, x.astype(jnp.int32))
